# Initial kernel scaffold; baseline (speedup 1.0000x reference)
import jax
import jax.numpy as jnp
from jax.experimental import pallas as pl


def kernel(per_atom_charge, atomic_subsystem_indices, pair_indices, d_ij):
    raise NotImplementedError("write your pallas kernel here")


# speedup vs baseline: 269.0292x; 269.0292x over previous
import functools

import jax
import jax.numpy as jnp
from jax import lax
from jax.experimental import pallas as pl
from jax.experimental.pallas import tpu as pltpu
from jax.experimental.pallas import tpu_sc as plsc

N_ATOMS = 100000
N_PAIRS = 6400000
CUTOFF = 5.0

TC_ROWS = 25000
TC_COLS = 256
TC_GRID = 25
TC_BLOCK_ROWS = TC_ROWS // TC_GRID

SC_INFO = plsc.get_sparse_core_info()
NUM_CORES = SC_INFO.num_cores
NUM_SUBCORES = SC_INFO.num_subcores
NUM_WORKERS = NUM_CORES * NUM_SUBCORES
PAIRS_PER_WORKER = N_PAIRS // NUM_WORKERS
CHUNK = 4000
NUM_CHUNKS = PAIRS_PER_WORKER // CHUNK
VECS_PER_CHUNK = CHUNK // 16


def _chi_body(d_ref, chi_ref, s_ref, acc_ref):
    d = d_ref[...]
    td = 2.0 * d
    phi = jnp.where(td < CUTOFF, 0.5 * (jnp.cos(jnp.pi * td / CUTOFF) + 1.0), 0.0)
    chi = phi * (1.0 / jnp.sqrt(d * d + 1.0)) + (1.0 - phi) * (1.0 / d)
    chi_ref[...] = chi
    ps = jnp.sum(chi)
    i = pl.program_id(0)

    @pl.when(i == 0)
    def _():
        acc_ref[0] = ps

    @pl.when(i > 0)
    def _():
        acc_ref[0] = acc_ref[0] + ps

    @pl.when(i == pl.num_programs(0) - 1)
    def _():
        s_ref[0] = acc_ref[0]


_chi_and_sum = pl.pallas_call(
    _chi_body,
    grid=(TC_GRID,),
    in_specs=[pl.BlockSpec((TC_BLOCK_ROWS, TC_COLS), lambda i: (i, 0))],
    out_specs=[
        pl.BlockSpec((TC_BLOCK_ROWS, TC_COLS), lambda i: (i, 0)),
        pl.BlockSpec(memory_space=pltpu.SMEM),
    ],
    out_shape=[
        jax.ShapeDtypeStruct((TC_ROWS, TC_COLS), jnp.float32),
        jax.ShapeDtypeStruct((1,), jnp.float32),
    ],
    scratch_shapes=[pltpu.SMEM((1,), jnp.float32)],
)


def _sc_body(charge_hbm, idx_hbm, chi_hbm, s_hbm, out_hbm,
             table_v, ii_v, jj_v, chi_v, out_v, s_v):
    wid = lax.axis_index("s") * NUM_CORES + lax.axis_index("c")
    pltpu.sync_copy(charge_hbm, table_v)
    pltpu.sync_copy(s_hbm, s_v)
    s_vec = s_v[...]
    base0 = wid * PAIRS_PER_WORKER

    def chunk_body(ci, carry):
        base = pl.multiple_of(base0 + ci * CHUNK, 8)
        pltpu.sync_copy(idx_hbm.at[pl.ds(base, CHUNK)], ii_v)
        pltpu.sync_copy(idx_hbm.at[pl.ds(N_PAIRS + base, CHUNK)], jj_v)
        pltpu.sync_copy(chi_hbm.at[pl.ds(base, CHUNK)], chi_v)

        def vec_body(k, carry2):
            o = pl.multiple_of(k * 16, 16)
            ii = ii_v[pl.ds(o, 16)]
            jj = jj_v[pl.ds(o, 16)]
            qi = plsc.load_gather(table_v, [ii])
            qj = plsc.load_gather(table_v, [jj])
            ch = chi_v[pl.ds(o, 16)]
            out_v[pl.ds(o, 16)] = qi * qj * (s_vec - ch)
            return carry2

        lax.fori_loop(0, VECS_PER_CHUNK, vec_body, 0)
        pltpu.sync_copy(out_v, out_hbm.at[pl.ds(base, CHUNK)])
        return carry

    lax.fori_loop(0, NUM_CHUNKS, chunk_body, 0)


_sc_combine = functools.partial(
    pl.kernel,
    out_type=jax.ShapeDtypeStruct((N_PAIRS,), jnp.float32),
    mesh=plsc.VectorSubcoreMesh(core_axis_name="c", subcore_axis_name="s"),
    compiler_params=pltpu.CompilerParams(needs_layout_passes=False),
    scratch_types=[
        pltpu.VMEM((N_ATOMS,), jnp.float32),
        pltpu.VMEM((CHUNK,), jnp.int32),
        pltpu.VMEM((CHUNK,), jnp.int32),
        pltpu.VMEM((CHUNK,), jnp.float32),
        pltpu.VMEM((CHUNK,), jnp.float32),
        pltpu.VMEM((16,), jnp.float32),
    ],
)(_sc_body)


def kernel(per_atom_charge, atomic_subsystem_indices, pair_indices, d_ij):
    del atomic_subsystem_indices
    charge = per_atom_charge.reshape(-1)
    idx_flat = pair_indices.astype(jnp.int32).reshape(-1)
    chi2d, s = _chi_and_sum(d_ij.reshape(TC_ROWS, TC_COLS))
    chi = chi2d.reshape(-1)
    s16 = jnp.broadcast_to(s, (16,))
    return _sc_combine(charge, idx_flat, chi, s16)


# speedup vs baseline: 365.2245x; 1.3576x over previous
import functools

import jax
import jax.numpy as jnp
from jax import lax
from jax.experimental import pallas as pl
from jax.experimental.pallas import tpu as pltpu
from jax.experimental.pallas import tpu_sc as plsc

N_ATOMS = 100000
N_PAIRS = 6400000
CUTOFF = 5.0

TC_GRID = 50
TC_BLOCK = N_PAIRS // TC_GRID

SC_INFO = plsc.get_sparse_core_info()
NUM_CORES = SC_INFO.num_cores
NUM_SUBCORES = SC_INFO.num_subcores
NUM_WORKERS = NUM_CORES * NUM_SUBCORES
PAIR_TILE = 128
N_TILES = N_PAIRS // PAIR_TILE
TILES_PER_CHUNK = 32
CHUNK = TILES_PER_CHUNK * PAIR_TILE
VECS_PER_CHUNK = CHUNK // 16
TILES_LO = N_TILES // NUM_WORKERS
TILES_EXTRA = N_TILES % NUM_WORKERS
NUM_CHUNKS = -(-(TILES_LO + 1) // TILES_PER_CHUNK)


def _chi(d):
    td = 2.0 * d
    phi = jnp.where(td < CUTOFF, 0.5 * (jnp.cos(jnp.pi * td / CUTOFF) + 1.0), 0.0)
    return phi * (1.0 / jnp.sqrt(d * d + 1.0)) + (1.0 - phi) * (1.0 / d)


def _sum_body(d_ref, s_ref, acc_ref):
    ps = jnp.sum(_chi(d_ref[...]))
    i = pl.program_id(0)

    @pl.when(i == 0)
    def _():
        acc_ref[0] = ps

    @pl.when(i > 0)
    def _():
        acc_ref[0] = acc_ref[0] + ps

    @pl.when(i == pl.num_programs(0) - 1)
    def _():
        s_ref[0] = acc_ref[0]


_chi_sum = pl.pallas_call(
    _sum_body,
    grid=(TC_GRID,),
    in_specs=[pl.BlockSpec((TC_BLOCK,), lambda i: (i,))],
    out_specs=pl.BlockSpec(memory_space=pltpu.SMEM),
    out_shape=jax.ShapeDtypeStruct((1,), jnp.float32),
    scratch_shapes=[pltpu.SMEM((1,), jnp.float32)],
)


def _combine_body(s_ref, qq_ref, d_ref, out_ref):
    s = s_ref[0]
    out_ref[...] = qq_ref[...] * (s - _chi(d_ref[...]))


_combine = pl.pallas_call(
    _combine_body,
    grid=(TC_GRID,),
    in_specs=[
        pl.BlockSpec(memory_space=pltpu.SMEM),
        pl.BlockSpec((TC_BLOCK,), lambda i: (i,)),
        pl.BlockSpec((TC_BLOCK,), lambda i: (i,)),
    ],
    out_specs=pl.BlockSpec((TC_BLOCK,), lambda i: (i,)),
    out_shape=jax.ShapeDtypeStruct((N_PAIRS,), jnp.float32),
)


def _sc_body(charge_hbm, idx_hbm, qq_hbm, table_v, ij_v, out_v):
    wid = lax.axis_index("s") * NUM_CORES + lax.axis_index("c")
    pltpu.sync_copy(charge_hbm, table_v)
    t0 = wid * TILES_LO + jnp.minimum(wid, TILES_EXTRA)
    ntiles = TILES_LO + jnp.where(wid < TILES_EXTRA, 1, 0)

    def chunk_body(ci, carry):
        tstart = t0 + jnp.minimum(ci * TILES_PER_CHUNK, ntiles - TILES_PER_CHUNK)
        base = pl.multiple_of(tstart * PAIR_TILE, PAIR_TILE)
        pltpu.sync_copy(idx_hbm.at[:, pl.ds(base, CHUNK)], ij_v)

        def vec_body(k, carry2):
            o = pl.multiple_of(k * 16, 16)
            qi = plsc.load_gather(table_v, [ij_v[0, pl.ds(o, 16)]])
            qj = plsc.load_gather(table_v, [ij_v[1, pl.ds(o, 16)]])
            out_v[pl.ds(o, 16)] = qi * qj
            return carry2

        lax.fori_loop(0, VECS_PER_CHUNK, vec_body, 0)
        pltpu.sync_copy(out_v, qq_hbm.at[pl.ds(base, CHUNK)])
        return carry

    lax.fori_loop(0, NUM_CHUNKS, chunk_body, 0)


_sc_qq = functools.partial(
    pl.kernel,
    out_type=jax.ShapeDtypeStruct((N_PAIRS,), jnp.float32),
    mesh=plsc.VectorSubcoreMesh(core_axis_name="c", subcore_axis_name="s"),
    compiler_params=pltpu.CompilerParams(needs_layout_passes=False),
    scratch_types=[
        pltpu.VMEM((N_ATOMS,), jnp.float32),
        pltpu.VMEM((2, CHUNK), jnp.int32),
        pltpu.VMEM((CHUNK,), jnp.float32),
    ],
)(_sc_body)


def kernel(per_atom_charge, atomic_subsystem_indices, pair_indices, d_ij):
    del atomic_subsystem_indices
    charge = per_atom_charge.reshape(-1)
    idx = pair_indices.astype(jnp.int32)
    qq = _sc_qq(charge, idx)
    s = _chi_sum(d_ij)
    return _combine(s, qq, d_ij)


# speedup vs baseline: 494.2762x; 1.3533x over previous
import functools

import jax
import jax.numpy as jnp
from jax import lax
from jax.experimental import pallas as pl
from jax.experimental.pallas import tpu as pltpu
from jax.experimental.pallas import tpu_sc as plsc

N_ATOMS = 100000
N_PAIRS = 6400000
CUTOFF = 5.0

TC_GRID = 50
TC_BLOCK = N_PAIRS // TC_GRID

SC_INFO = plsc.get_sparse_core_info()
NUM_CORES = SC_INFO.num_cores
NUM_SUBCORES = SC_INFO.num_subcores
NUM_WORKERS = NUM_CORES * NUM_SUBCORES
PAIR_TILE = 128
N_TILES = N_PAIRS // PAIR_TILE
TILES_PER_CHUNK = 64
CHUNK = TILES_PER_CHUNK * PAIR_TILE
TILES_LO = N_TILES // NUM_WORKERS
TILES_EXTRA = N_TILES % NUM_WORKERS
NUM_CHUNKS = -(-(TILES_LO + 1) // TILES_PER_CHUNK)


def _chi(d):
    td = 2.0 * d
    phi = jnp.where(td < CUTOFF, 0.5 * (jnp.cos(jnp.pi * td / CUTOFF) + 1.0), 0.0)
    return phi * (1.0 / jnp.sqrt(d * d + 1.0)) + (1.0 - phi) * (1.0 / d)


def _sum_body(d_ref, s_ref, acc_ref):
    ps = jnp.sum(_chi(d_ref[...]))
    i = pl.program_id(0)

    @pl.when(i == 0)
    def _():
        acc_ref[0] = ps

    @pl.when(i > 0)
    def _():
        acc_ref[0] = acc_ref[0] + ps

    @pl.when(i == pl.num_programs(0) - 1)
    def _():
        s_ref[0] = acc_ref[0]


_chi_sum = pl.pallas_call(
    _sum_body,
    grid=(TC_GRID,),
    in_specs=[pl.BlockSpec((TC_BLOCK,), lambda i: (i,))],
    out_specs=pl.BlockSpec(memory_space=pltpu.SMEM),
    out_shape=jax.ShapeDtypeStruct((1,), jnp.float32),
    scratch_shapes=[pltpu.SMEM((1,), jnp.float32)],
)


def _combine_body(s_ref, qq_ref, d_ref, out_ref):
    s = s_ref[0]
    out_ref[...] = qq_ref[...] * (s - _chi(d_ref[...]))


_combine = pl.pallas_call(
    _combine_body,
    grid=(TC_GRID,),
    in_specs=[
        pl.BlockSpec(memory_space=pltpu.SMEM),
        pl.BlockSpec((TC_BLOCK,), lambda i: (i,)),
        pl.BlockSpec((TC_BLOCK,), lambda i: (i,)),
    ],
    out_specs=pl.BlockSpec((TC_BLOCK,), lambda i: (i,)),
    out_shape=jax.ShapeDtypeStruct((N_PAIRS,), jnp.float32),
)


def _sc_body(charge_hbm, idx_hbm, qq_hbm, table_v, ij_v, out_v):
    wid = lax.axis_index("s") * NUM_CORES + lax.axis_index("c")
    pltpu.sync_copy(charge_hbm, table_v)
    t0 = wid * TILES_LO + jnp.minimum(wid, TILES_EXTRA)
    ntiles = TILES_LO + jnp.where(wid < TILES_EXTRA, 1, 0)

    def chunk_body(ci, carry):
        tstart = t0 + jnp.minimum(ci * TILES_PER_CHUNK, ntiles - TILES_PER_CHUNK)
        base = pl.multiple_of(tstart * PAIR_TILE, PAIR_TILE)
        pltpu.sync_copy(idx_hbm.at[:, pl.ds(base, CHUNK)], ij_v)

        @plsc.parallel_loop(0, CHUNK, step=16, unroll=8)
        def _(o):
            qi = plsc.load_gather(table_v, [ij_v[0, pl.ds(o, 16)]])
            qj = plsc.load_gather(table_v, [ij_v[1, pl.ds(o, 16)]])
            out_v[pl.ds(o, 16)] = qi * qj

        pltpu.sync_copy(out_v, qq_hbm.at[pl.ds(base, CHUNK)])
        return carry

    lax.fori_loop(0, NUM_CHUNKS, chunk_body, 0)


_sc_qq = functools.partial(
    pl.kernel,
    out_type=jax.ShapeDtypeStruct((N_PAIRS,), jnp.float32),
    mesh=plsc.VectorSubcoreMesh(core_axis_name="c", subcore_axis_name="s"),
    compiler_params=pltpu.CompilerParams(needs_layout_passes=False),
    scratch_types=[
        pltpu.VMEM((N_ATOMS,), jnp.float32),
        pltpu.VMEM((2, CHUNK), jnp.int32),
        pltpu.VMEM((CHUNK,), jnp.float32),
    ],
)(_sc_body)


def kernel(per_atom_charge, atomic_subsystem_indices, pair_indices, d_ij):
    del atomic_subsystem_indices
    charge = per_atom_charge.reshape(-1)
    idx = pair_indices.astype(jnp.int32)
    qq = _sc_qq(charge, idx)
    s = _chi_sum(d_ij)
    return _combine(s, qq, d_ij)


# speedup vs baseline: 550.1682x; 1.1131x over previous
import functools

import jax
import jax.numpy as jnp
from jax import lax
from jax.experimental import pallas as pl
from jax.experimental.pallas import tpu as pltpu
from jax.experimental.pallas import tpu_sc as plsc

N_ATOMS = 100000
N_PAIRS = 6400000
CUTOFF = 5.0

TC_GRID = 50
TC_BLOCK = N_PAIRS // TC_GRID

SC_INFO = plsc.get_sparse_core_info()
NUM_CORES = SC_INFO.num_cores
NUM_SUBCORES = SC_INFO.num_subcores
NUM_WORKERS = NUM_CORES * NUM_SUBCORES
PAIR_TILE = 128
N_TILES = N_PAIRS // PAIR_TILE
TILES_PER_CHUNK = 64
CHUNK = TILES_PER_CHUNK * PAIR_TILE
TILES_LO = N_TILES // NUM_WORKERS
TILES_EXTRA = N_TILES % NUM_WORKERS
NUM_CHUNKS = -(-(TILES_LO + 1) // TILES_PER_CHUNK)


def _chi(d):
    td = 2.0 * d
    phi = jnp.where(td < CUTOFF, 0.5 * (jnp.cos(jnp.pi * td / CUTOFF) + 1.0), 0.0)
    return phi * (1.0 / jnp.sqrt(d * d + 1.0)) + (1.0 - phi) * (1.0 / d)


def _sum_body(d_ref, chi_ref, s_ref, acc_ref):
    chi = _chi(d_ref[...])
    chi_ref[...] = chi
    ps = jnp.sum(chi)
    i = pl.program_id(0)

    @pl.when(i == 0)
    def _():
        acc_ref[0] = ps

    @pl.when(i > 0)
    def _():
        acc_ref[0] = acc_ref[0] + ps

    @pl.when(i == pl.num_programs(0) - 1)
    def _():
        s_ref[0] = acc_ref[0]


_chi_sum = pl.pallas_call(
    _sum_body,
    grid=(TC_GRID,),
    in_specs=[pl.BlockSpec((TC_BLOCK,), lambda i: (i,))],
    out_specs=[
        pl.BlockSpec((TC_BLOCK,), lambda i: (i,)),
        pl.BlockSpec(memory_space=pltpu.SMEM),
    ],
    out_shape=[
        jax.ShapeDtypeStruct((N_PAIRS,), jnp.float32),
        jax.ShapeDtypeStruct((1,), jnp.float32),
    ],
    scratch_shapes=[pltpu.SMEM((1,), jnp.float32)],
)


def _combine_body(s_ref, qq_ref, chi_ref, out_ref):
    s = s_ref[0]
    out_ref[...] = qq_ref[...] * (s - chi_ref[...])


_combine = pl.pallas_call(
    _combine_body,
    grid=(TC_GRID,),
    in_specs=[
        pl.BlockSpec(memory_space=pltpu.SMEM),
        pl.BlockSpec((TC_BLOCK,), lambda i: (i,)),
        pl.BlockSpec((TC_BLOCK,), lambda i: (i,)),
    ],
    out_specs=pl.BlockSpec((TC_BLOCK,), lambda i: (i,)),
    out_shape=jax.ShapeDtypeStruct((N_PAIRS,), jnp.float32),
)


def _sc_body(charge_hbm, idx_hbm, qq_hbm, table_v, ij_v, out_v):
    wid = lax.axis_index("s") * NUM_CORES + lax.axis_index("c")
    pltpu.sync_copy(charge_hbm, table_v)
    t0 = wid * TILES_LO + jnp.minimum(wid, TILES_EXTRA)
    ntiles = TILES_LO + jnp.where(wid < TILES_EXTRA, 1, 0)

    def chunk_body(ci, carry):
        tstart = t0 + jnp.minimum(ci * TILES_PER_CHUNK, ntiles - TILES_PER_CHUNK)
        base = pl.multiple_of(tstart * PAIR_TILE, PAIR_TILE)
        pltpu.sync_copy(idx_hbm.at[:, pl.ds(base, CHUNK)], ij_v)

        @plsc.parallel_loop(0, CHUNK, step=16, unroll=8)
        def _(o):
            qi = plsc.load_gather(table_v, [ij_v[0, pl.ds(o, 16)]])
            qj = plsc.load_gather(table_v, [ij_v[1, pl.ds(o, 16)]])
            out_v[pl.ds(o, 16)] = qi * qj

        pltpu.sync_copy(out_v, qq_hbm.at[pl.ds(base, CHUNK)])
        return carry

    lax.fori_loop(0, NUM_CHUNKS, chunk_body, 0)


_sc_qq = functools.partial(
    pl.kernel,
    out_type=jax.ShapeDtypeStruct((N_PAIRS,), jnp.float32),
    mesh=plsc.VectorSubcoreMesh(core_axis_name="c", subcore_axis_name="s"),
    compiler_params=pltpu.CompilerParams(needs_layout_passes=False),
    scratch_types=[
        pltpu.VMEM((N_ATOMS,), jnp.float32),
        pltpu.VMEM((2, CHUNK), jnp.int32),
        pltpu.VMEM((CHUNK,), jnp.float32),
    ],
)(_sc_body)


def kernel(per_atom_charge, atomic_subsystem_indices, pair_indices, d_ij):
    del atomic_subsystem_indices
    charge = per_atom_charge.reshape(-1)
    idx = pair_indices.astype(jnp.int32)
    qq = _sc_qq(charge, idx)
    chi, s = _chi_sum(d_ij)
    return _combine(s, qq, chi)


# speedup vs baseline: 776.1230x; 1.4107x over previous
import functools

import jax
import jax.numpy as jnp
from jax import lax
from jax.experimental import pallas as pl
from jax.experimental.pallas import tpu as pltpu
from jax.experimental.pallas import tpu_sc as plsc

N_ATOMS = 100000
N_PAIRS = 6400000
CUTOFF = 5.0

TC_GRID = 50
TC_BLOCK = N_PAIRS // TC_GRID

SC_INFO = plsc.get_sparse_core_info()
NUM_CORES = SC_INFO.num_cores
NUM_SUBCORES = SC_INFO.num_subcores
NUM_WORKERS = NUM_CORES * NUM_SUBCORES
PAIR_TILE = 128
N_TILES = N_PAIRS // PAIR_TILE
TILES_PER_CHUNK = 64
CHUNK = TILES_PER_CHUNK * PAIR_TILE
TILES_LO = N_TILES // NUM_WORKERS
TILES_EXTRA = N_TILES % NUM_WORKERS
NUM_CHUNKS = -(-(TILES_LO + 1) // TILES_PER_CHUNK)


def _chi(d):
    u = (jnp.pi / CUTOFF) * d
    v = u * u
    c = 0.99999528 + v * (-0.49993092 + v * (0.04151173 + v * (-0.00127871)))
    phi = jnp.where(d < 0.5 * CUTOFF, c * c, 0.0)
    return phi * lax.rsqrt(d * d + 1.0) + (1.0 - phi) / d


def _sum_body(d_ref, chi_ref, s_ref, acc_ref):
    chi = _chi(d_ref[...])
    chi_ref[...] = chi
    ps = jnp.sum(chi)
    i = pl.program_id(0)

    @pl.when(i == 0)
    def _():
        acc_ref[0] = ps

    @pl.when(i > 0)
    def _():
        acc_ref[0] = acc_ref[0] + ps

    @pl.when(i == pl.num_programs(0) - 1)
    def _():
        s_ref[0] = acc_ref[0]


_chi_sum = pl.pallas_call(
    _sum_body,
    grid=(TC_GRID,),
    in_specs=[pl.BlockSpec((TC_BLOCK,), lambda i: (i,))],
    out_specs=[
        pl.BlockSpec((TC_BLOCK,), lambda i: (i,)),
        pl.BlockSpec(memory_space=pltpu.SMEM),
    ],
    out_shape=[
        jax.ShapeDtypeStruct((N_PAIRS,), jnp.float32),
        jax.ShapeDtypeStruct((1,), jnp.float32),
    ],
    scratch_shapes=[pltpu.SMEM((1,), jnp.float32)],
)


def _combine_body(s_ref, qq_ref, chi_ref, out_ref):
    s = s_ref[0]
    out_ref[...] = qq_ref[...] * (s - chi_ref[...])


_combine = pl.pallas_call(
    _combine_body,
    grid=(TC_GRID,),
    in_specs=[
        pl.BlockSpec(memory_space=pltpu.SMEM),
        pl.BlockSpec((TC_BLOCK,), lambda i: (i,)),
        pl.BlockSpec((TC_BLOCK,), lambda i: (i,)),
    ],
    out_specs=pl.BlockSpec((TC_BLOCK,), lambda i: (i,)),
    out_shape=jax.ShapeDtypeStruct((N_PAIRS,), jnp.float32),
)


def _sc_body(charge_hbm, idx_hbm, qq_hbm, table_v, ij_v, out_v):
    wid = lax.axis_index("s") * NUM_CORES + lax.axis_index("c")
    pltpu.sync_copy(charge_hbm, table_v)
    t0 = wid * TILES_LO + jnp.minimum(wid, TILES_EXTRA)
    ntiles = TILES_LO + jnp.where(wid < TILES_EXTRA, 1, 0)

    def chunk_body(ci, carry):
        tstart = t0 + jnp.minimum(ci * TILES_PER_CHUNK, ntiles - TILES_PER_CHUNK)
        base = pl.multiple_of(tstart * PAIR_TILE, PAIR_TILE)
        pltpu.sync_copy(idx_hbm.at[:, pl.ds(base, CHUNK)], ij_v)

        @plsc.parallel_loop(0, CHUNK, step=16, unroll=8)
        def _(o):
            qi = plsc.load_gather(table_v, [ij_v[0, pl.ds(o, 16)]])
            qj = plsc.load_gather(table_v, [ij_v[1, pl.ds(o, 16)]])
            out_v[pl.ds(o, 16)] = qi * qj

        pltpu.sync_copy(out_v, qq_hbm.at[pl.ds(base, CHUNK)])
        return carry

    lax.fori_loop(0, NUM_CHUNKS, chunk_body, 0)


_sc_qq = functools.partial(
    pl.kernel,
    out_type=jax.ShapeDtypeStruct((N_PAIRS,), jnp.float32),
    mesh=plsc.VectorSubcoreMesh(core_axis_name="c", subcore_axis_name="s"),
    compiler_params=pltpu.CompilerParams(needs_layout_passes=False),
    scratch_types=[
        pltpu.VMEM((N_ATOMS,), jnp.float32),
        pltpu.VMEM((2, CHUNK), jnp.int32),
        pltpu.VMEM((CHUNK,), jnp.float32),
    ],
)(_sc_body)


def kernel(per_atom_charge, atomic_subsystem_indices, pair_indices, d_ij):
    del atomic_subsystem_indices
    charge = per_atom_charge.reshape(-1)
    idx = pair_indices.astype(jnp.int32)
    qq = _sc_qq(charge, idx)
    chi, s = _chi_sum(d_ij)
    return _combine(s, qq, chi)


# speedup vs baseline: 831.0908x; 1.0708x over previous
import functools

import jax
import jax.numpy as jnp
from jax import lax
from jax.experimental import pallas as pl
from jax.experimental.pallas import tpu as pltpu
from jax.experimental.pallas import tpu_sc as plsc

N_ATOMS = 100000
N_PAIRS = 6400000
CUTOFF = 5.0

TC_GRID = 50
TC_BLOCK = N_PAIRS // TC_GRID

SC_INFO = plsc.get_sparse_core_info()
NUM_CORES = SC_INFO.num_cores
NUM_SUBCORES = SC_INFO.num_subcores
NUM_WORKERS = NUM_CORES * NUM_SUBCORES
PAIR_TILE = 128
N_TILES = N_PAIRS // PAIR_TILE
TILES_PER_CHUNK = 32
CHUNK = TILES_PER_CHUNK * PAIR_TILE
TILES_LO = N_TILES // NUM_WORKERS
TILES_EXTRA = N_TILES % NUM_WORKERS
NUM_CHUNKS = 50


def _chi(d):
    u = (jnp.pi / CUTOFF) * d
    v = u * u
    c = 0.99999528 + v * (-0.49993092 + v * (0.04151173 + v * (-0.00127871)))
    phi = jnp.where(d < 0.5 * CUTOFF, c * c, 0.0)
    return phi * lax.rsqrt(d * d + 1.0) + (1.0 - phi) / d


def _sum_body(d_ref, chi_ref, s_ref, acc_ref):
    chi = _chi(d_ref[...])
    chi_ref[...] = chi
    ps = jnp.sum(chi)
    i = pl.program_id(0)

    @pl.when(i == 0)
    def _():
        acc_ref[0] = ps

    @pl.when(i > 0)
    def _():
        acc_ref[0] = acc_ref[0] + ps

    @pl.when(i == pl.num_programs(0) - 1)
    def _():
        s_ref[0] = acc_ref[0]


_chi_sum = pl.pallas_call(
    _sum_body,
    grid=(TC_GRID,),
    in_specs=[pl.BlockSpec((TC_BLOCK,), lambda i: (i,))],
    out_specs=[
        pl.BlockSpec((TC_BLOCK,), lambda i: (i,)),
        pl.BlockSpec(memory_space=pltpu.SMEM),
    ],
    out_shape=[
        jax.ShapeDtypeStruct((N_PAIRS,), jnp.float32),
        jax.ShapeDtypeStruct((1,), jnp.float32),
    ],
    scratch_shapes=[pltpu.SMEM((1,), jnp.float32)],
)


def _combine_body(s_ref, qq_ref, chi_ref, out_ref):
    s = s_ref[0]
    out_ref[...] = qq_ref[...] * (s - chi_ref[...])


_combine = pl.pallas_call(
    _combine_body,
    grid=(TC_GRID,),
    in_specs=[
        pl.BlockSpec(memory_space=pltpu.SMEM),
        pl.BlockSpec((TC_BLOCK,), lambda i: (i,)),
        pl.BlockSpec((TC_BLOCK,), lambda i: (i,)),
    ],
    out_specs=pl.BlockSpec((TC_BLOCK,), lambda i: (i,)),
    out_shape=jax.ShapeDtypeStruct((N_PAIRS,), jnp.float32),
)


def _sc_body(charge_hbm, idx_hbm, qq_hbm, table_v, ij_v, out_v,
             si0, si1, so0, so1):
    wid = lax.axis_index("s") * NUM_CORES + lax.axis_index("c")
    pltpu.sync_copy(charge_hbm, table_v)
    t0 = wid * TILES_LO + jnp.minimum(wid, TILES_EXTRA)
    ntiles = TILES_LO + jnp.where(wid < TILES_EXTRA, 1, 0)
    sin = (si0, si1)
    sout = (so0, so1)

    def base_of(ci):
        tstart = t0 + jnp.minimum(ci * TILES_PER_CHUNK, ntiles - TILES_PER_CHUNK)
        return pl.multiple_of(tstart * PAIR_TILE, PAIR_TILE)

    def in_copy(ci, b):
        return pltpu.make_async_copy(
            idx_hbm.at[:, pl.ds(base_of(ci), CHUNK)], ij_v.at[b], sin[b])

    def out_copy(ci, b):
        return pltpu.make_async_copy(
            out_v.at[b], qq_hbm.at[pl.ds(base_of(ci), CHUNK)], sout[b])

    in_copy(0, 0).start()

    def group(g, carry):
        for b in (0, 1):
            c = 2 * g + b
            nb = 1 - b

            @pl.when(c + 1 < NUM_CHUNKS)
            def _():
                in_copy(c + 1, nb).start()

            in_copy(c, b).wait()

            @pl.when(c >= 2)
            def _():
                out_copy(c - 2, b).wait()

            @plsc.parallel_loop(0, CHUNK, step=16, unroll=8)
            def _(o):
                qi = plsc.load_gather(table_v, [ij_v[b, 0, pl.ds(o, 16)]])
                qj = plsc.load_gather(table_v, [ij_v[b, 1, pl.ds(o, 16)]])
                out_v[b, pl.ds(o, 16)] = qi * qj

            out_copy(c, b).start()
        return carry

    lax.fori_loop(0, NUM_CHUNKS // 2, group, 0)
    out_copy(NUM_CHUNKS - 2, 0).wait()
    out_copy(NUM_CHUNKS - 1, 1).wait()


_sc_qq = functools.partial(
    pl.kernel,
    out_type=jax.ShapeDtypeStruct((N_PAIRS,), jnp.float32),
    mesh=plsc.VectorSubcoreMesh(core_axis_name="c", subcore_axis_name="s"),
    compiler_params=pltpu.CompilerParams(needs_layout_passes=False),
    scratch_types=[
        pltpu.VMEM((N_ATOMS,), jnp.float32),
        pltpu.VMEM((2, 2, CHUNK), jnp.int32),
        pltpu.VMEM((2, CHUNK), jnp.float32),
        pltpu.SemaphoreType.DMA,
        pltpu.SemaphoreType.DMA,
        pltpu.SemaphoreType.DMA,
        pltpu.SemaphoreType.DMA,
    ],
)(_sc_body)


def kernel(per_atom_charge, atomic_subsystem_indices, pair_indices, d_ij):
    del atomic_subsystem_indices
    charge = per_atom_charge.reshape(-1)
    idx = pair_indices.astype(jnp.int32)
    qq = _sc_qq(charge, idx)
    chi, s = _chi_sum(d_ij)
    return _combine(s, qq, chi)


# speedup vs baseline: 835.4504x; 1.0052x over previous
import functools

import jax
import jax.numpy as jnp
from jax import lax
from jax.experimental import pallas as pl
from jax.experimental.pallas import tpu as pltpu
from jax.experimental.pallas import tpu_sc as plsc

N_ATOMS = 100000
N_PAIRS = 6400000
CUTOFF = 5.0

TC_GRID = 50
TC_BLOCK = N_PAIRS // TC_GRID

SC_INFO = plsc.get_sparse_core_info()
NUM_CORES = SC_INFO.num_cores
NUM_SUBCORES = SC_INFO.num_subcores
NUM_WORKERS = NUM_CORES * NUM_SUBCORES
PAIR_TILE = 128
N_TILES = N_PAIRS // PAIR_TILE
TILES_PER_CHUNK = 32
CHUNK = TILES_PER_CHUNK * PAIR_TILE
TILES_LO = N_TILES // NUM_WORKERS
TILES_EXTRA = N_TILES % NUM_WORKERS
NUM_CHUNKS = 50


def _chi(d):
    r2 = d * d
    v = (jnp.pi / CUTOFF) * (jnp.pi / CUTOFF) * r2
    c = 0.99999528 + v * (-0.49993092 + v * (0.04151173 + v * (-0.00127871)))
    phi = jnp.where(d < 0.5 * CUTOFF, c * c, 0.0)
    inv_d = lax.rsqrt(r2)
    return inv_d + phi * (lax.rsqrt(r2 + 1.0) - inv_d)


def _sum_body(d_ref, chi_ref, s_ref, acc_ref):
    chi = _chi(d_ref[...])
    chi_ref[...] = chi
    ps = jnp.sum(chi)
    i = pl.program_id(0)

    @pl.when(i == 0)
    def _():
        acc_ref[0] = ps

    @pl.when(i > 0)
    def _():
        acc_ref[0] = acc_ref[0] + ps

    @pl.when(i == pl.num_programs(0) - 1)
    def _():
        s_ref[0] = acc_ref[0]


_chi_sum = pl.pallas_call(
    _sum_body,
    grid=(TC_GRID,),
    in_specs=[pl.BlockSpec((TC_BLOCK,), lambda i: (i,))],
    out_specs=[
        pl.BlockSpec((TC_BLOCK,), lambda i: (i,)),
        pl.BlockSpec(memory_space=pltpu.SMEM),
    ],
    out_shape=[
        jax.ShapeDtypeStruct((N_PAIRS,), jnp.float32),
        jax.ShapeDtypeStruct((1,), jnp.float32),
    ],
    scratch_shapes=[pltpu.SMEM((1,), jnp.float32)],
)


def _combine_body(s_ref, qq_ref, chi_ref, out_ref):
    s = s_ref[0]
    out_ref[...] = qq_ref[...] * (s - chi_ref[...])


_combine = pl.pallas_call(
    _combine_body,
    grid=(TC_GRID,),
    in_specs=[
        pl.BlockSpec(memory_space=pltpu.SMEM),
        pl.BlockSpec((TC_BLOCK,), lambda i: (i,)),
        pl.BlockSpec((TC_BLOCK,), lambda i: (i,)),
    ],
    out_specs=pl.BlockSpec((TC_BLOCK,), lambda i: (i,)),
    out_shape=jax.ShapeDtypeStruct((N_PAIRS,), jnp.float32),
)


def _sc_body(charge_hbm, idx_hbm, qq_hbm, table_v, ij_v, out_v,
             si0, si1, so0, so1):
    wid = lax.axis_index("s") * NUM_CORES + lax.axis_index("c")
    pltpu.sync_copy(charge_hbm, table_v)
    t0 = wid * TILES_LO + jnp.minimum(wid, TILES_EXTRA)
    ntiles = TILES_LO + jnp.where(wid < TILES_EXTRA, 1, 0)
    sin = (si0, si1)
    sout = (so0, so1)

    def base_of(ci):
        tstart = t0 + jnp.minimum(ci * TILES_PER_CHUNK, ntiles - TILES_PER_CHUNK)
        return pl.multiple_of(tstart * PAIR_TILE, PAIR_TILE)

    def in_copy(ci, b):
        return pltpu.make_async_copy(
            idx_hbm.at[:, pl.ds(base_of(ci), CHUNK)], ij_v.at[b], sin[b])

    def out_copy(ci, b):
        return pltpu.make_async_copy(
            out_v.at[b], qq_hbm.at[pl.ds(base_of(ci), CHUNK)], sout[b])

    in_copy(0, 0).start()

    def group(g, carry):
        for b in (0, 1):
            c = 2 * g + b
            nb = 1 - b

            @pl.when(c + 1 < NUM_CHUNKS)
            def _():
                in_copy(c + 1, nb).start()

            in_copy(c, b).wait()

            @pl.when(c >= 2)
            def _():
                out_copy(c - 2, b).wait()

            @plsc.parallel_loop(0, CHUNK, step=16, unroll=16)
            def _(o):
                qi = plsc.load_gather(table_v, [ij_v[b, 0, pl.ds(o, 16)]])
                qj = plsc.load_gather(table_v, [ij_v[b, 1, pl.ds(o, 16)]])
                out_v[b, pl.ds(o, 16)] = qi * qj

            out_copy(c, b).start()
        return carry

    lax.fori_loop(0, NUM_CHUNKS // 2, group, 0)
    out_copy(NUM_CHUNKS - 2, 0).wait()
    out_copy(NUM_CHUNKS - 1, 1).wait()


_sc_qq = functools.partial(
    pl.kernel,
    out_type=jax.ShapeDtypeStruct((N_PAIRS,), jnp.float32),
    mesh=plsc.VectorSubcoreMesh(core_axis_name="c", subcore_axis_name="s"),
    compiler_params=pltpu.CompilerParams(needs_layout_passes=False),
    scratch_types=[
        pltpu.VMEM((N_ATOMS,), jnp.float32),
        pltpu.VMEM((2, 2, CHUNK), jnp.int32),
        pltpu.VMEM((2, CHUNK), jnp.float32),
        pltpu.SemaphoreType.DMA,
        pltpu.SemaphoreType.DMA,
        pltpu.SemaphoreType.DMA,
        pltpu.SemaphoreType.DMA,
    ],
)(_sc_body)


def kernel(per_atom_charge, atomic_subsystem_indices, pair_indices, d_ij):
    del atomic_subsystem_indices
    charge = per_atom_charge.reshape(-1)
    idx = pair_indices.astype(jnp.int32)
    qq = _sc_qq(charge, idx)
    chi, s = _chi_sum(d_ij)
    return _combine(s, qq, chi)


# speedup vs baseline: 928.9741x; 1.1119x over previous
import functools

import jax
import jax.numpy as jnp
from jax import lax
from jax.experimental import pallas as pl
from jax.experimental.pallas import tpu as pltpu
from jax.experimental.pallas import tpu_sc as plsc

N_ATOMS = 100000
N_PAIRS = 6400000
CUTOFF = 5.0

TC_GRID = 50
TC_BLOCK = N_PAIRS // TC_GRID

SC_INFO = plsc.get_sparse_core_info()
NUM_CORES = SC_INFO.num_cores
NUM_SUBCORES = SC_INFO.num_subcores
NUM_WORKERS = NUM_CORES * NUM_SUBCORES
PAIR_TILE = 128
N_TILES = N_PAIRS // PAIR_TILE
TILES_PER_CHUNK = 32
CHUNK = TILES_PER_CHUNK * PAIR_TILE
TILES_LO = N_TILES // NUM_WORKERS
TILES_EXTRA = N_TILES % NUM_WORKERS
NUM_CHUNKS = 50


def _chi(d):
    r2 = d * d
    v = (jnp.pi / CUTOFF) * (jnp.pi / CUTOFF) * r2
    c = 0.99999528 + v * (-0.49993092 + v * (0.04151173 + v * (-0.00127871)))
    phi = jnp.where(d < 0.5 * CUTOFF, c * c, 0.0)
    inv_d = lax.rsqrt(r2)
    return inv_d + phi * (lax.rsqrt(r2 + 1.0) - inv_d)


def _sum_body(d_ref, chi_ref, s_ref, acc_ref):
    chi = _chi(d_ref[...])
    chi_ref[...] = chi
    i = pl.program_id(0)

    @pl.when(i == 0)
    def _():
        acc_ref[...] = chi

    @pl.when(i > 0)
    def _():
        acc_ref[...] = acc_ref[...] + chi

    @pl.when(i == pl.num_programs(0) - 1)
    def _():
        s_ref[0] = jnp.sum(acc_ref[...])


_chi_sum = pl.pallas_call(
    _sum_body,
    grid=(TC_GRID,),
    in_specs=[pl.BlockSpec((TC_BLOCK,), lambda i: (i,))],
    out_specs=[
        pl.BlockSpec((TC_BLOCK,), lambda i: (i,)),
        pl.BlockSpec(memory_space=pltpu.SMEM),
    ],
    out_shape=[
        jax.ShapeDtypeStruct((N_PAIRS,), jnp.float32),
        jax.ShapeDtypeStruct((1,), jnp.float32),
    ],
    scratch_shapes=[pltpu.VMEM((TC_BLOCK,), jnp.float32)],
)


def _combine_body(s_ref, qq_ref, chi_ref, out_ref):
    s = s_ref[0]
    out_ref[...] = qq_ref[...] * (s - chi_ref[...])


_combine = pl.pallas_call(
    _combine_body,
    grid=(TC_GRID,),
    in_specs=[
        pl.BlockSpec(memory_space=pltpu.SMEM),
        pl.BlockSpec((TC_BLOCK,), lambda i: (i,)),
        pl.BlockSpec((TC_BLOCK,), lambda i: (i,)),
    ],
    out_specs=pl.BlockSpec((TC_BLOCK,), lambda i: (i,)),
    out_shape=jax.ShapeDtypeStruct((N_PAIRS,), jnp.float32),
)


def _sc_body(charge_hbm, idx_hbm, qq_hbm, table_v, ij_v, out_v,
             si0, si1, so0, so1):
    wid = lax.axis_index("s") * NUM_CORES + lax.axis_index("c")
    pltpu.sync_copy(charge_hbm, table_v)
    t0 = wid * TILES_LO + jnp.minimum(wid, TILES_EXTRA)
    ntiles = TILES_LO + jnp.where(wid < TILES_EXTRA, 1, 0)
    sin = (si0, si1)
    sout = (so0, so1)

    def base_of(ci):
        tstart = t0 + jnp.minimum(ci * TILES_PER_CHUNK, ntiles - TILES_PER_CHUNK)
        return pl.multiple_of(tstart * PAIR_TILE, PAIR_TILE)

    def in_copy(ci, b):
        return pltpu.make_async_copy(
            idx_hbm.at[:, pl.ds(base_of(ci), CHUNK)], ij_v.at[b], sin[b])

    def out_copy(ci, b):
        return pltpu.make_async_copy(
            out_v.at[b], qq_hbm.at[pl.ds(base_of(ci), CHUNK)], sout[b])

    in_copy(0, 0).start()

    def group(g, carry):
        for b in (0, 1):
            c = 2 * g + b
            nb = 1 - b

            @pl.when(c + 1 < NUM_CHUNKS)
            def _():
                in_copy(c + 1, nb).start()

            in_copy(c, b).wait()

            @pl.when(c >= 2)
            def _():
                out_copy(c - 2, b).wait()

            @plsc.parallel_loop(0, CHUNK, step=16, unroll=16)
            def _(o):
                qi = plsc.load_gather(table_v, [ij_v[b, 0, pl.ds(o, 16)]])
                qj = plsc.load_gather(table_v, [ij_v[b, 1, pl.ds(o, 16)]])
                out_v[b, pl.ds(o, 16)] = qi * qj

            out_copy(c, b).start()
        return carry

    lax.fori_loop(0, NUM_CHUNKS // 2, group, 0)
    out_copy(NUM_CHUNKS - 2, 0).wait()
    out_copy(NUM_CHUNKS - 1, 1).wait()


_sc_qq = functools.partial(
    pl.kernel,
    out_type=jax.ShapeDtypeStruct((N_PAIRS,), jnp.float32),
    mesh=plsc.VectorSubcoreMesh(core_axis_name="c", subcore_axis_name="s"),
    compiler_params=pltpu.CompilerParams(needs_layout_passes=False),
    scratch_types=[
        pltpu.VMEM((N_ATOMS,), jnp.float32),
        pltpu.VMEM((2, 2, CHUNK), jnp.int32),
        pltpu.VMEM((2, CHUNK), jnp.float32),
        pltpu.SemaphoreType.DMA,
        pltpu.SemaphoreType.DMA,
        pltpu.SemaphoreType.DMA,
        pltpu.SemaphoreType.DMA,
    ],
)(_sc_body)


def kernel(per_atom_charge, atomic_subsystem_indices, pair_indices, d_ij):
    del atomic_subsystem_indices
    charge = per_atom_charge.reshape(-1)
    idx = pair_indices.astype(jnp.int32)
    qq = _sc_qq(charge, idx)
    chi, s = _chi_sum(d_ij)
    return _combine(s, qq, chi)


# speedup vs baseline: 1014.7717x; 1.0924x over previous
import functools

import jax
import jax.numpy as jnp
from jax import lax
from jax.experimental import pallas as pl
from jax.experimental.pallas import tpu as pltpu
from jax.experimental.pallas import tpu_sc as plsc

N_ATOMS = 100000
N_PAIRS = 6400000
CUTOFF = 5.0

TC_GRID = 25
TC_BLOCK = N_PAIRS // TC_GRID

SC_INFO = plsc.get_sparse_core_info()
NUM_CORES = SC_INFO.num_cores
NUM_SUBCORES = SC_INFO.num_subcores
NUM_WORKERS = NUM_CORES * NUM_SUBCORES
PAIR_TILE = 128
N_TILES = N_PAIRS // PAIR_TILE
TILES_PER_CHUNK = 32
CHUNK = TILES_PER_CHUNK * PAIR_TILE
TILES_LO = N_TILES // NUM_WORKERS
TILES_EXTRA = N_TILES % NUM_WORKERS
NUM_CHUNKS = 50


def _chi(d):
    r2 = d * d
    v = (jnp.pi / CUTOFF) * (jnp.pi / CUTOFF) * r2
    c = 0.99999528 + v * (-0.49993092 + v * (0.04151173 + v * (-0.00127871)))
    phi = jnp.where(d < 0.5 * CUTOFF, c * c, 0.0)
    inv_d = lax.rsqrt(r2)
    return inv_d + phi * (lax.rsqrt(r2 + 1.0) - inv_d)


def _sum_body(d_ref, chi_ref, s_ref, acc_ref):
    chi = _chi(d_ref[...])
    chi_ref[...] = chi
    i = pl.program_id(0)

    @pl.when(i == 0)
    def _():
        acc_ref[...] = chi

    @pl.when(i > 0)
    def _():
        acc_ref[...] = acc_ref[...] + chi

    @pl.when(i == pl.num_programs(0) - 1)
    def _():
        s_ref[0] = jnp.sum(acc_ref[...])


_chi_sum = pl.pallas_call(
    _sum_body,
    grid=(TC_GRID,),
    in_specs=[pl.BlockSpec((TC_BLOCK,), lambda i: (i,))],
    out_specs=[
        pl.BlockSpec((TC_BLOCK,), lambda i: (i,)),
        pl.BlockSpec(memory_space=pltpu.SMEM),
    ],
    out_shape=[
        jax.ShapeDtypeStruct((N_PAIRS,), jnp.float32),
        jax.ShapeDtypeStruct((1,), jnp.float32),
    ],
    scratch_shapes=[pltpu.VMEM((TC_BLOCK,), jnp.float32)],
)


def _combine_body(s_ref, qq_ref, chi_ref, out_ref):
    s = s_ref[0]
    out_ref[...] = qq_ref[...] * (s - chi_ref[...])


_combine = pl.pallas_call(
    _combine_body,
    grid=(TC_GRID,),
    in_specs=[
        pl.BlockSpec(memory_space=pltpu.SMEM),
        pl.BlockSpec((TC_BLOCK,), lambda i: (i,)),
        pl.BlockSpec((TC_BLOCK,), lambda i: (i,)),
    ],
    out_specs=pl.BlockSpec((TC_BLOCK,), lambda i: (i,)),
    out_shape=jax.ShapeDtypeStruct((N_PAIRS,), jnp.float32),
)


def _sc_body(charge_hbm, idx_hbm, qq_hbm, table_v, ij_v, out_v,
             si0, si1, so0, so1):
    wid = lax.axis_index("s") * NUM_CORES + lax.axis_index("c")
    pltpu.sync_copy(charge_hbm, table_v)
    t0 = wid * TILES_LO + jnp.minimum(wid, TILES_EXTRA)
    ntiles = TILES_LO + jnp.where(wid < TILES_EXTRA, 1, 0)
    sin = (si0, si1)
    sout = (so0, so1)

    def base_of(ci):
        tstart = t0 + jnp.minimum(ci * TILES_PER_CHUNK, ntiles - TILES_PER_CHUNK)
        return pl.multiple_of(tstart * PAIR_TILE, PAIR_TILE)

    def in_copy(ci, b):
        return pltpu.make_async_copy(
            idx_hbm.at[:, pl.ds(base_of(ci), CHUNK)], ij_v.at[b], sin[b])

    def out_copy(ci, b):
        return pltpu.make_async_copy(
            out_v.at[b], qq_hbm.at[pl.ds(base_of(ci), CHUNK)], sout[b])

    in_copy(0, 0).start()

    def group(g, carry):
        for b in (0, 1):
            c = 2 * g + b
            nb = 1 - b

            @pl.when(c + 1 < NUM_CHUNKS)
            def _():
                in_copy(c + 1, nb).start()

            in_copy(c, b).wait()

            @pl.when(c >= 2)
            def _():
                out_copy(c - 2, b).wait()

            @plsc.parallel_loop(0, CHUNK, step=16, unroll=32)
            def _(o):
                qi = plsc.load_gather(table_v, [ij_v[b, 0, pl.ds(o, 16)]])
                qj = plsc.load_gather(table_v, [ij_v[b, 1, pl.ds(o, 16)]])
                out_v[b, pl.ds(o, 16)] = qi * qj

            out_copy(c, b).start()
        return carry

    lax.fori_loop(0, NUM_CHUNKS // 2, group, 0)
    out_copy(NUM_CHUNKS - 2, 0).wait()
    out_copy(NUM_CHUNKS - 1, 1).wait()


_sc_qq = functools.partial(
    pl.kernel,
    out_type=jax.ShapeDtypeStruct((N_PAIRS,), jnp.float32),
    mesh=plsc.VectorSubcoreMesh(core_axis_name="c", subcore_axis_name="s"),
    compiler_params=pltpu.CompilerParams(needs_layout_passes=False),
    scratch_types=[
        pltpu.VMEM((N_ATOMS,), jnp.float32),
        pltpu.VMEM((2, 2, CHUNK), jnp.int32),
        pltpu.VMEM((2, CHUNK), jnp.float32),
        pltpu.SemaphoreType.DMA,
        pltpu.SemaphoreType.DMA,
        pltpu.SemaphoreType.DMA,
        pltpu.SemaphoreType.DMA,
    ],
)(_sc_body)


def kernel(per_atom_charge, atomic_subsystem_indices, pair_indices, d_ij):
    del atomic_subsystem_indices
    charge = per_atom_charge.reshape(-1)
    idx = pair_indices.astype(jnp.int32)
    qq = _sc_qq(charge, idx)
    chi, s = _chi_sum(d_ij)
    return _combine(s, qq, chi)


# speedup vs baseline: 1043.2103x; 1.0280x over previous
import functools

import jax
import jax.numpy as jnp
from jax import lax
from jax.experimental import pallas as pl
from jax.experimental.pallas import tpu as pltpu
from jax.experimental.pallas import tpu_sc as plsc

N_ATOMS = 100000
N_PAIRS = 6400000
CUTOFF = 5.0

TC_GRID = 25
TC_BLOCK = N_PAIRS // TC_GRID

SC_INFO = plsc.get_sparse_core_info()
NUM_CORES = SC_INFO.num_cores
NUM_SUBCORES = SC_INFO.num_subcores
NUM_WORKERS = NUM_CORES * NUM_SUBCORES
PAIR_TILE = 128
N_TILES = N_PAIRS // PAIR_TILE
TILES_PER_CHUNK = 40
CHUNK = TILES_PER_CHUNK * PAIR_TILE
TILES_LO = N_TILES // NUM_WORKERS
TILES_EXTRA = N_TILES % NUM_WORKERS
NUM_CHUNKS = 40


def _chi(d):
    r2 = d * d
    v = (jnp.pi / CUTOFF) * (jnp.pi / CUTOFF) * r2
    c = 0.99999528 + v * (-0.49993092 + v * (0.04151173 + v * (-0.00127871)))
    phi = jnp.where(d < 0.5 * CUTOFF, c * c, 0.0)
    inv_d = lax.rsqrt(r2)
    return inv_d + phi * (lax.rsqrt(r2 + 1.0) - inv_d)


def _sum_body(d_ref, chi_ref, s_ref, acc_ref):
    chi = _chi(d_ref[...])
    chi_ref[...] = chi
    i = pl.program_id(0)

    @pl.when(i == 0)
    def _():
        acc_ref[...] = chi

    @pl.when(i > 0)
    def _():
        acc_ref[...] = acc_ref[...] + chi

    @pl.when(i == pl.num_programs(0) - 1)
    def _():
        s_ref[0] = jnp.sum(acc_ref[...])


_chi_sum = pl.pallas_call(
    _sum_body,
    grid=(TC_GRID,),
    in_specs=[pl.BlockSpec((TC_BLOCK,), lambda i: (i,))],
    out_specs=[
        pl.BlockSpec((TC_BLOCK,), lambda i: (i,)),
        pl.BlockSpec(memory_space=pltpu.SMEM),
    ],
    out_shape=[
        jax.ShapeDtypeStruct((N_PAIRS,), jnp.float32),
        jax.ShapeDtypeStruct((1,), jnp.float32),
    ],
    scratch_shapes=[pltpu.VMEM((TC_BLOCK,), jnp.float32)],
)


def _combine_body(s_ref, qq_ref, chi_ref, out_ref):
    s = s_ref[0]
    out_ref[...] = qq_ref[...] * (s - chi_ref[...])


_combine = pl.pallas_call(
    _combine_body,
    grid=(TC_GRID,),
    in_specs=[
        pl.BlockSpec(memory_space=pltpu.SMEM),
        pl.BlockSpec((TC_BLOCK,), lambda i: (i,)),
        pl.BlockSpec((TC_BLOCK,), lambda i: (i,)),
    ],
    out_specs=pl.BlockSpec((TC_BLOCK,), lambda i: (i,)),
    out_shape=jax.ShapeDtypeStruct((N_PAIRS,), jnp.float32),
)


def _sc_body(charge_hbm, idx_hbm, qq_hbm, table_v, ij_v, out_v,
             si0, si1, so0, so1):
    wid = lax.axis_index("s") * NUM_CORES + lax.axis_index("c")
    t0 = wid * TILES_LO + jnp.minimum(wid, TILES_EXTRA)
    ntiles = TILES_LO + jnp.where(wid < TILES_EXTRA, 1, 0)
    sin = (si0, si1)
    sout = (so0, so1)

    def base_of(ci):
        tstart = t0 + jnp.minimum(ci * TILES_PER_CHUNK, ntiles - TILES_PER_CHUNK)
        return pl.multiple_of(tstart * PAIR_TILE, PAIR_TILE)

    def in_copy(ci, b):
        return pltpu.make_async_copy(
            idx_hbm.at[:, pl.ds(base_of(ci), CHUNK)], ij_v.at[b], sin[b])

    def out_copy(ci, b):
        return pltpu.make_async_copy(
            out_v.at[b], qq_hbm.at[pl.ds(base_of(ci), CHUNK)], sout[b])

    in_copy(0, 0).start()
    pltpu.sync_copy(charge_hbm, table_v)

    def group(g, carry):
        for b in (0, 1):
            c = 2 * g + b
            nb = 1 - b

            @pl.when(c + 1 < NUM_CHUNKS)
            def _():
                in_copy(c + 1, nb).start()

            in_copy(c, b).wait()

            @pl.when(c >= 2)
            def _():
                out_copy(c - 2, b).wait()

            @plsc.parallel_loop(0, CHUNK, step=16, unroll=16)
            def _(o):
                qi = plsc.load_gather(table_v, [ij_v[b, 0, pl.ds(o, 16)]])
                qj = plsc.load_gather(table_v, [ij_v[b, 1, pl.ds(o, 16)]])
                out_v[b, pl.ds(o, 16)] = qi * qj

            out_copy(c, b).start()
        return carry

    lax.fori_loop(0, NUM_CHUNKS // 2, group, 0)
    out_copy(NUM_CHUNKS - 2, 0).wait()
    out_copy(NUM_CHUNKS - 1, 1).wait()


_sc_qq = functools.partial(
    pl.kernel,
    out_type=jax.ShapeDtypeStruct((N_PAIRS,), jnp.float32),
    mesh=plsc.VectorSubcoreMesh(core_axis_name="c", subcore_axis_name="s"),
    compiler_params=pltpu.CompilerParams(needs_layout_passes=False),
    scratch_types=[
        pltpu.VMEM((N_ATOMS,), jnp.float32),
        pltpu.VMEM((2, 2, CHUNK), jnp.int32),
        pltpu.VMEM((2, CHUNK), jnp.float32),
        pltpu.SemaphoreType.DMA,
        pltpu.SemaphoreType.DMA,
        pltpu.SemaphoreType.DMA,
        pltpu.SemaphoreType.DMA,
    ],
)(_sc_body)


def kernel(per_atom_charge, atomic_subsystem_indices, pair_indices, d_ij):
    del atomic_subsystem_indices
    charge = per_atom_charge.reshape(-1)
    idx = pair_indices.astype(jnp.int32)
    qq = _sc_qq(charge, idx)
    chi, s = _chi_sum(d_ij)
    return _combine(s, qq, chi)


# speedup vs baseline: 1078.5436x; 1.0339x over previous
import functools

import jax
import jax.numpy as jnp
from jax import lax
from jax.experimental import pallas as pl
from jax.experimental.pallas import tpu as pltpu
from jax.experimental.pallas import tpu_sc as plsc

N_ATOMS = 100000
N_PAIRS = 6400000
CUTOFF = 5.0

TC_GRID = 10
TC_BLOCK = N_PAIRS // TC_GRID

SC_INFO = plsc.get_sparse_core_info()
NUM_CORES = SC_INFO.num_cores
NUM_SUBCORES = SC_INFO.num_subcores
NUM_WORKERS = NUM_CORES * NUM_SUBCORES
PAIR_TILE = 128
N_TILES = N_PAIRS // PAIR_TILE
TILES_PER_CHUNK = 40
CHUNK = TILES_PER_CHUNK * PAIR_TILE
TILES_LO = N_TILES // NUM_WORKERS
TILES_EXTRA = N_TILES % NUM_WORKERS
NUM_CHUNKS = 40


def _chi(d):
    r2 = d * d
    v = (jnp.pi / CUTOFF) * (jnp.pi / CUTOFF) * r2
    c = 0.99999528 + v * (-0.49993092 + v * (0.04151173 + v * (-0.00127871)))
    phi = jnp.where(d < 0.5 * CUTOFF, c * c, 0.0)
    inv_d = lax.rsqrt(r2)
    return inv_d + phi * (lax.rsqrt(r2 + 1.0) - inv_d)


def _sum_body(d_ref, chi_ref, s_ref, acc_ref):
    chi = _chi(d_ref[...])
    chi_ref[...] = chi
    i = pl.program_id(0)

    @pl.when(i == 0)
    def _():
        acc_ref[...] = chi

    @pl.when(i > 0)
    def _():
        acc_ref[...] = acc_ref[...] + chi

    @pl.when(i == pl.num_programs(0) - 1)
    def _():
        s_ref[0] = jnp.sum(acc_ref[...])


_chi_sum = pl.pallas_call(
    _sum_body,
    grid=(TC_GRID,),
    in_specs=[pl.BlockSpec((TC_BLOCK,), lambda i: (i,))],
    out_specs=[
        pl.BlockSpec((TC_BLOCK,), lambda i: (i,)),
        pl.BlockSpec(memory_space=pltpu.SMEM),
    ],
    out_shape=[
        jax.ShapeDtypeStruct((N_PAIRS,), jnp.float32),
        jax.ShapeDtypeStruct((1,), jnp.float32),
    ],
    scratch_shapes=[pltpu.VMEM((TC_BLOCK,), jnp.float32)],
)


def _combine_body(s_ref, qq_ref, chi_ref, out_ref):
    s = s_ref[0]
    out_ref[...] = qq_ref[...] * (s - chi_ref[...])


_combine = pl.pallas_call(
    _combine_body,
    grid=(TC_GRID,),
    in_specs=[
        pl.BlockSpec(memory_space=pltpu.SMEM),
        pl.BlockSpec((TC_BLOCK,), lambda i: (i,)),
        pl.BlockSpec((TC_BLOCK,), lambda i: (i,)),
    ],
    out_specs=pl.BlockSpec((TC_BLOCK,), lambda i: (i,)),
    out_shape=jax.ShapeDtypeStruct((N_PAIRS,), jnp.float32),
)


def _sc_body(charge_hbm, idx_hbm, qq_hbm, table_v, ij_v, out_v,
             si0, si1, so0, so1):
    wid = lax.axis_index("s") * NUM_CORES + lax.axis_index("c")
    t0 = wid * TILES_LO + jnp.minimum(wid, TILES_EXTRA)
    ntiles = TILES_LO + jnp.where(wid < TILES_EXTRA, 1, 0)
    sin = (si0, si1)
    sout = (so0, so1)

    def base_of(ci):
        tstart = t0 + jnp.minimum(ci * TILES_PER_CHUNK, ntiles - TILES_PER_CHUNK)
        return pl.multiple_of(tstart * PAIR_TILE, PAIR_TILE)

    def in_copy(ci, b):
        return pltpu.make_async_copy(
            idx_hbm.at[:, pl.ds(base_of(ci), CHUNK)], ij_v.at[b], sin[b])

    def out_copy(ci, b):
        return pltpu.make_async_copy(
            out_v.at[b], qq_hbm.at[pl.ds(base_of(ci), CHUNK)], sout[b])

    in_copy(0, 0).start()
    pltpu.sync_copy(charge_hbm, table_v)

    def group(g, carry):
        for b in (0, 1):
            c = 2 * g + b
            nb = 1 - b

            @pl.when(c + 1 < NUM_CHUNKS)
            def _():
                in_copy(c + 1, nb).start()

            in_copy(c, b).wait()

            @pl.when(c >= 2)
            def _():
                out_copy(c - 2, b).wait()

            @plsc.parallel_loop(0, CHUNK, step=16, unroll=16)
            def _(o):
                qi = plsc.load_gather(table_v, [ij_v[b, 0, pl.ds(o, 16)]])
                qj = plsc.load_gather(table_v, [ij_v[b, 1, pl.ds(o, 16)]])
                out_v[b, pl.ds(o, 16)] = qi * qj

            out_copy(c, b).start()
        return carry

    lax.fori_loop(0, NUM_CHUNKS // 2, group, 0)
    out_copy(NUM_CHUNKS - 2, 0).wait()
    out_copy(NUM_CHUNKS - 1, 1).wait()


_sc_qq = functools.partial(
    pl.kernel,
    out_type=jax.ShapeDtypeStruct((N_PAIRS,), jnp.float32),
    mesh=plsc.VectorSubcoreMesh(core_axis_name="c", subcore_axis_name="s"),
    compiler_params=pltpu.CompilerParams(needs_layout_passes=False),
    scratch_types=[
        pltpu.VMEM((N_ATOMS,), jnp.float32),
        pltpu.VMEM((2, 2, CHUNK), jnp.int32),
        pltpu.VMEM((2, CHUNK), jnp.float32),
        pltpu.SemaphoreType.DMA,
        pltpu.SemaphoreType.DMA,
        pltpu.SemaphoreType.DMA,
        pltpu.SemaphoreType.DMA,
    ],
)(_sc_body)


def kernel(per_atom_charge, atomic_subsystem_indices, pair_indices, d_ij):
    del atomic_subsystem_indices
    charge = per_atom_charge.reshape(-1)
    idx = pair_indices.astype(jnp.int32)
    qq = _sc_qq(charge, idx)
    chi, s = _chi_sum(d_ij)
    return _combine(s, qq, chi)
